# baseline (device time: 85973 ns/iter reference)
import jax
import jax.numpy as jnp
from jax import lax
from jax.experimental import pallas as pl
from jax.experimental.pallas import tpu as pltpu

N_DEV = 4
S = 2048
H = 8
DH = 128
DM = H * DH
W = 128
EXT = S + 2 * W
BQ = 256
BK = BQ + 2 * W
NBLK = S // BQ
SCALE = 0.08838834764831843
NEG = -1e9


def kernel(x, Wq, K_ext, V_ext, Wo):
    x2 = x[0].astype(jnp.bfloat16)
    k2 = K_ext[0].reshape(S, DM)
    v2 = V_ext[0].reshape(S, DM)
    wq = Wq.astype(jnp.bfloat16)
    wo = Wo.astype(jnp.bfloat16)

    def body(x_ref, wq_ref, k_ref, v_ref, wo_ref, out_ref,
             ext_k, ext_v, stage, q_scr, ctx_scr, mask_scr,
             send_sems, recv_sems):
        p = lax.axis_index("i")
        left = lax.rem(p + N_DEV - 1, N_DEV)
        right = lax.rem(p + 1, N_DEV)

        barrier = pltpu.get_barrier_semaphore()
        for nbr in (left, right):
            pl.semaphore_signal(barrier, inc=1, device_id=(nbr,),
                                device_id_type=pl.DeviceIdType.MESH)
        pl.semaphore_wait(barrier, 2)

        stage[0, :, :] = k_ref[0:W, :].astype(jnp.bfloat16)
        stage[1, :, :] = k_ref[S - W:S, :].astype(jnp.bfloat16)
        stage[2, :, :] = v_ref[0:W, :].astype(jnp.bfloat16)
        stage[3, :, :] = v_ref[S - W:S, :].astype(jnp.bfloat16)

        plan = [
            (0, ext_k, S + W, left),
            (1, ext_k, 0, right),
            (2, ext_v, S + W, left),
            (3, ext_v, 0, right),
        ]
        rdmas = []
        for i, (slot, dst, row0, tgt) in enumerate(plan):
            r = pltpu.make_async_remote_copy(
                src_ref=stage.at[slot],
                dst_ref=dst.at[pl.ds(row0, W), :],
                send_sem=send_sems.at[i],
                recv_sem=recv_sems.at[i],
                device_id=(tgt,),
                device_id_type=pl.DeviceIdType.MESH,
            )
            r.start()
            rdmas.append(r)

        def kv_step(rb, _):
            r0 = rb * BQ
            ext_k[pl.ds(W + r0, BQ), :] = (
                k_ref[pl.ds(r0, BQ), :].astype(jnp.bfloat16))
            ext_v[pl.ds(W + r0, BQ), :] = (
                v_ref[pl.ds(r0, BQ), :].astype(jnp.bfloat16))
            return 0

        lax.fori_loop(0, NBLK, kv_step, 0)

        def q_step(rb, _):
            r0 = rb * BQ
            qv = lax.dot(x_ref[pl.ds(r0, BQ), :], wq_ref[:, :],
                         preferred_element_type=jnp.float32)
            q_scr[pl.ds(r0, BQ), :] = (qv * SCALE).astype(jnp.bfloat16)
            return 0

        lax.fori_loop(0, NBLK, q_step, 0)

        ii = lax.broadcasted_iota(jnp.int32, (BQ, BK), 0)
        jj = lax.broadcasted_iota(jnp.int32, (BQ, BK), 1)
        band = jnp.where((jj >= ii) & (jj - ii <= 2 * W), 0.0, NEG)
        is_first = (p == 0)
        is_last = (p == N_DEV - 1)
        j_last_thresh = S + W - (NBLK - 1) * BQ
        mask_scr[0, :, :] = band
        mask_scr[1, :, :] = band + jnp.where(is_first & (jj < W), NEG, 0.0)
        mask_scr[2, :, :] = band + jnp.where(
            is_last & (jj >= j_last_thresh), NEG, 0.0)

        for r in rdmas:
            r.wait()

        for h in range(H):
            c0 = h * DH

            def attn_step(qb, _, c0=c0):
                r0 = qb * BQ
                qblk = q_scr[pl.ds(r0, BQ), c0:c0 + DH]
                kblk = ext_k[pl.ds(r0, BK), c0:c0 + DH]
                vblk = ext_v[pl.ds(r0, BK), c0:c0 + DH]

                sel = jnp.where(qb == 0, 1,
                                jnp.where(qb == NBLK - 1, 2, 0))
                s = lax.dot_general(
                    qblk, kblk, (((1,), (1,)), ((), ())),
                    preferred_element_type=jnp.float32)
                s = s + mask_scr[sel, :, :]

                e = jnp.exp(s)
                denom = jnp.sum(e, axis=1, keepdims=True)
                ctx = lax.dot_general(
                    e.astype(jnp.bfloat16), vblk,
                    (((1,), (0,)), ((), ())),
                    preferred_element_type=jnp.float32)
                ctx = ctx / denom
                ctx_scr[pl.ds(r0, BQ), c0:c0 + DH] = ctx.astype(jnp.bfloat16)
                return 0

            lax.fori_loop(0, NBLK, attn_step, 0)

        def out_step(rb, _):
            r0 = rb * BQ
            out_ref[pl.ds(r0, BQ), :] = lax.dot(
                ctx_scr[pl.ds(r0, BQ), :], wo_ref[:, :],
                preferred_element_type=jnp.float32)
            return 0

        lax.fori_loop(0, NBLK, out_step, 0)

    out = pl.pallas_call(
        body,
        out_shape=jax.ShapeDtypeStruct((S, DM), jnp.float32),
        in_specs=[pl.BlockSpec(memory_space=pltpu.VMEM)] * 5,
        out_specs=pl.BlockSpec(memory_space=pltpu.VMEM),
        scratch_shapes=[
            pltpu.VMEM((EXT, DM), jnp.bfloat16),
            pltpu.VMEM((EXT, DM), jnp.bfloat16),
            pltpu.VMEM((4, W, DM), jnp.bfloat16),
            pltpu.VMEM((S, DM), jnp.bfloat16),
            pltpu.VMEM((S, DM), jnp.bfloat16),
            pltpu.VMEM((3, BQ, BK), jnp.float32),
            pltpu.SemaphoreType.DMA((4,)),
            pltpu.SemaphoreType.DMA((4,)),
        ],
        compiler_params=pltpu.CompilerParams(
            collective_id=0,
            vmem_limit_bytes=60 * 1024 * 1024,
        ),
    )(x2, wq, k2, v2, wo)

    return out.reshape(1, S, DM)


# device time: 83150 ns/iter; 1.0340x vs baseline; 1.0340x over previous
import jax
import jax.numpy as jnp
from jax import lax
from jax.experimental import pallas as pl
from jax.experimental.pallas import tpu as pltpu

N_DEV = 4
S = 2048
H = 8
DH = 128
DM = H * DH
W = 128
BQ = 256
BK = BQ + 2 * W
NBLK = S // BQ
SCALE = 0.08838834764831843
NEG = -1e9


def kernel(x, Wq, K_ext, V_ext, Wo):
    x2 = x[0].astype(jnp.bfloat16)
    k2 = K_ext[0].reshape(S, DM).astype(jnp.bfloat16)
    v2 = V_ext[0].reshape(S, DM).astype(jnp.bfloat16)
    wq = (Wq * SCALE).astype(jnp.bfloat16)
    wo = Wo.astype(jnp.bfloat16)

    def body(x_ref, wq_ref, k_ref, v_ref, wo_ref, out_ref,
             edge_k, edge_v, q_scr, mask_scr,
             send_sems, recv_sems, copy_sems):
        p = lax.axis_index("i")
        left = lax.rem(p + N_DEV - 1, N_DEV)
        right = lax.rem(p + 1, N_DEV)

        barrier = pltpu.get_barrier_semaphore()
        for nbr in (left, right):
            pl.semaphore_signal(barrier, inc=1, device_id=(nbr,),
                                device_id_type=pl.DeviceIdType.MESH,)
        pl.semaphore_wait(barrier, 2)

        plan = [
            (k_ref.at[pl.ds(0, W), :], edge_k.at[1, pl.ds(BK - W, W), :], left),
            (k_ref.at[pl.ds(S - W, W), :], edge_k.at[0, pl.ds(0, W), :], right),
            (v_ref.at[pl.ds(0, W), :], edge_v.at[1, pl.ds(BK - W, W), :], left),
            (v_ref.at[pl.ds(S - W, W), :], edge_v.at[0, pl.ds(0, W), :], right),
        ]
        rdmas = []
        for i, (src, dst, tgt) in enumerate(plan):
            r = pltpu.make_async_remote_copy(
                src_ref=src, dst_ref=dst,
                send_sem=send_sems.at[i], recv_sem=recv_sems.at[i],
                device_id=(tgt,), device_id_type=pl.DeviceIdType.MESH,
            )
            r.start()
            rdmas.append(r)

        lcopies = []
        lplan = [
            (k_ref.at[pl.ds(0, BK - W), :], edge_k.at[0, pl.ds(W, BK - W), :]),
            (k_ref.at[pl.ds(S - BK + W, BK - W), :], edge_k.at[1, pl.ds(0, BK - W), :]),
            (v_ref.at[pl.ds(0, BK - W), :], edge_v.at[0, pl.ds(W, BK - W), :]),
            (v_ref.at[pl.ds(S - BK + W, BK - W), :], edge_v.at[1, pl.ds(0, BK - W), :]),
        ]
        for i, (src, dst) in enumerate(lplan):
            c = pltpu.make_async_copy(src, dst, copy_sems.at[i])
            c.start()
            lcopies.append(c)

        def q_step(rb, _):
            r0 = rb * BQ
            qv = lax.dot(x_ref[pl.ds(r0, BQ), :], wq_ref[:, :],
                         preferred_element_type=jnp.float32)
            q_scr[pl.ds(r0, BQ), :] = qv.astype(jnp.bfloat16)
            return 0

        lax.fori_loop(0, NBLK, q_step, 0)

        ii = lax.broadcasted_iota(jnp.int32, (BQ, BK), 0)
        jj = lax.broadcasted_iota(jnp.int32, (BQ, BK), 1)
        band = jnp.where((jj >= ii) & (jj - ii <= 2 * W), 0.0, NEG)
        is_first = (p == 0)
        is_last = (p == N_DEV - 1)
        mask_scr[0, :, :] = band
        mask_scr[1, :, :] = band + jnp.where(is_first & (jj < W), NEG, 0.0)
        mask_scr[2, :, :] = band + jnp.where(is_last & (jj >= BK - W), NEG, 0.0)

        for c in lcopies:
            c.wait()
        for r in rdmas:
            r.wait()

        ones = jnp.ones((BK, DH), jnp.bfloat16)

        def attn_block(r0, kblk, vblk, mask, c0):
            qblk = q_scr[pl.ds(r0, BQ), c0:c0 + DH]
            s = lax.dot_general(
                qblk, kblk, (((1,), (1,)), ((), ())),
                preferred_element_type=jnp.float32)
            e = jnp.exp(s + mask).astype(jnp.bfloat16)
            denom = lax.dot_general(
                e, ones, (((1,), (0,)), ((), ())),
                preferred_element_type=jnp.float32)
            ctx = lax.dot_general(
                e, vblk, (((1,), (0,)), ((), ())),
                preferred_element_type=jnp.float32)
            q_scr[pl.ds(r0, BQ), c0:c0 + DH] = (ctx / denom).astype(jnp.bfloat16)

        for h in range(H):
            c0 = h * DH

            attn_block(0, edge_k[0, :, c0:c0 + DH], edge_v[0, :, c0:c0 + DH],
                       mask_scr[1, :, :], c0)

            def inner_step(qb, _, c0=c0):
                r0 = qb * BQ
                attn_block(r0,
                           k_ref[pl.ds(r0 - W, BK), c0:c0 + DH],
                           v_ref[pl.ds(r0 - W, BK), c0:c0 + DH],
                           mask_scr[0, :, :], c0)
                return 0

            lax.fori_loop(1, NBLK - 1, inner_step, 0)

            attn_block((NBLK - 1) * BQ,
                       edge_k[1, :, c0:c0 + DH], edge_v[1, :, c0:c0 + DH],
                       mask_scr[2, :, :], c0)

        def out_step(rb, _):
            r0 = rb * BQ
            out_ref[pl.ds(r0, BQ), :] = lax.dot(
                q_scr[pl.ds(r0, BQ), :], wo_ref[:, :],
                preferred_element_type=jnp.float32)
            return 0

        lax.fori_loop(0, NBLK, out_step, 0)

    out = pl.pallas_call(
        body,
        out_shape=jax.ShapeDtypeStruct((S, DM), jnp.float32),
        in_specs=[pl.BlockSpec(memory_space=pltpu.VMEM)] * 5,
        out_specs=pl.BlockSpec(memory_space=pltpu.VMEM),
        scratch_shapes=[
            pltpu.VMEM((2, BK, DM), jnp.bfloat16),
            pltpu.VMEM((2, BK, DM), jnp.bfloat16),
            pltpu.VMEM((S, DM), jnp.bfloat16),
            pltpu.VMEM((3, BQ, BK), jnp.float32),
            pltpu.SemaphoreType.DMA((4,)),
            pltpu.SemaphoreType.DMA((4,)),
            pltpu.SemaphoreType.DMA((4,)),
        ],
        compiler_params=pltpu.CompilerParams(
            collective_id=0,
            vmem_limit_bytes=60 * 1024 * 1024,
        ),
    )(x2, wq, k2, v2, wo)

    return out.reshape(1, S, DM)


# device time: 74190 ns/iter; 1.1588x vs baseline; 1.1208x over previous
import jax
import jax.numpy as jnp
from jax import lax
from jax.experimental import pallas as pl
from jax.experimental.pallas import tpu as pltpu

N_DEV = 4
S = 2048
H = 8
DH = 128
DM = H * DH
W = 128
BQ = 256
BK = BQ + 2 * W
NBLK = S // BQ
SCALE = 0.08838834764831843
NEG = -1e9


def kernel(x, Wq, K_ext, V_ext, Wo):
    x2 = x[0].astype(jnp.bfloat16)
    k2 = K_ext[0].reshape(S, DM).astype(jnp.bfloat16)
    v2 = V_ext[0].reshape(S, DM).astype(jnp.bfloat16)
    wq = (Wq * SCALE).astype(jnp.bfloat16)
    wo = Wo.astype(jnp.bfloat16)

    def body(x_ref, wq_ref, k_ref, v_ref, wo_ref, out_ref,
             edge_k, edge_v, q_scr, mask_scr,
             send_sems, recv_sems, copy_sems):
        p = lax.axis_index("i")
        left = lax.rem(p + N_DEV - 1, N_DEV)
        right = lax.rem(p + 1, N_DEV)

        barrier = pltpu.get_barrier_semaphore()
        for nbr in (left, right):
            pl.semaphore_signal(barrier, inc=1, device_id=(nbr,),
                                device_id_type=pl.DeviceIdType.MESH,)
        pl.semaphore_wait(barrier, 2)

        plan = [
            (k_ref.at[pl.ds(0, W), :], edge_k.at[1, pl.ds(BK - W, W), :], left),
            (k_ref.at[pl.ds(S - W, W), :], edge_k.at[0, pl.ds(0, W), :], right),
            (v_ref.at[pl.ds(0, W), :], edge_v.at[1, pl.ds(BK - W, W), :], left),
            (v_ref.at[pl.ds(S - W, W), :], edge_v.at[0, pl.ds(0, W), :], right),
        ]
        rdmas = []
        for i, (src, dst, tgt) in enumerate(plan):
            r = pltpu.make_async_remote_copy(
                src_ref=src, dst_ref=dst,
                send_sem=send_sems.at[i], recv_sem=recv_sems.at[i],
                device_id=(tgt,), device_id_type=pl.DeviceIdType.MESH,
            )
            r.start()
            rdmas.append(r)

        lcopies = []
        lplan = [
            (k_ref.at[pl.ds(0, BK - W), :], edge_k.at[0, pl.ds(W, BK - W), :]),
            (k_ref.at[pl.ds(S - BK + W, BK - W), :], edge_k.at[1, pl.ds(0, BK - W), :]),
            (v_ref.at[pl.ds(0, BK - W), :], edge_v.at[0, pl.ds(W, BK - W), :]),
            (v_ref.at[pl.ds(S - BK + W, BK - W), :], edge_v.at[1, pl.ds(0, BK - W), :]),
        ]
        for i, (src, dst) in enumerate(lplan):
            c = pltpu.make_async_copy(src, dst, copy_sems.at[i])
            c.start()
            lcopies.append(c)

        def q_step(rb, _):
            r0 = rb * BQ
            qv = lax.dot(x_ref[pl.ds(r0, BQ), :], wq_ref[:, :],
                         preferred_element_type=jnp.float32)
            q_scr[pl.ds(r0, BQ), :] = qv.astype(jnp.bfloat16)
            return 0

        lax.fori_loop(0, NBLK, q_step, 0)

        ii = lax.broadcasted_iota(jnp.int32, (BQ, BK), 0)
        jj = lax.broadcasted_iota(jnp.int32, (BQ, BK), 1)
        band = jnp.where((jj >= ii) & (jj - ii <= 2 * W), 0.0, NEG)
        is_first = (p == 0)
        is_last = (p == N_DEV - 1)
        mask_scr[0, :, :] = band
        mask_scr[1, :, :] = band + jnp.where(is_first & (jj < W), NEG, 0.0)
        mask_scr[2, :, :] = band + jnp.where(is_last & (jj >= BK - W), NEG, 0.0)

        for c in lcopies:
            c.wait()
        for r in rdmas:
            r.wait()

        ones = jnp.ones((BK, DH), jnp.bfloat16)

        def attn_block(r0, kblk, vblk, mask, c0):
            qblk = q_scr[pl.ds(r0, BQ), c0:c0 + DH]
            s = lax.dot_general(
                qblk, kblk, (((1,), (1,)), ((), ())),
                preferred_element_type=jnp.float32)
            e = jnp.exp(s + mask).astype(jnp.bfloat16)
            denom = lax.dot_general(
                e, ones, (((1,), (0,)), ((), ())),
                preferred_element_type=jnp.float32)
            ctx = lax.dot_general(
                e, vblk, (((1,), (0,)), ((), ())),
                preferred_element_type=jnp.float32)
            q_scr[pl.ds(r0, BQ), c0:c0 + DH] = (ctx / denom).astype(jnp.bfloat16)

        for h in range(H):
            c0 = h * DH

            attn_block(0, edge_k[0, :, c0:c0 + DH], edge_v[0, :, c0:c0 + DH],
                       mask_scr[1, :, :], c0)

            for qb in range(1, NBLK - 1):
                r0 = qb * BQ
                attn_block(r0,
                           k_ref[pl.ds(r0 - W, BK), c0:c0 + DH],
                           v_ref[pl.ds(r0 - W, BK), c0:c0 + DH],
                           mask_scr[0, :, :], c0)

            attn_block((NBLK - 1) * BQ,
                       edge_k[1, :, c0:c0 + DH], edge_v[1, :, c0:c0 + DH],
                       mask_scr[2, :, :], c0)

        def out_step(rb, _):
            r0 = rb * BQ
            out_ref[pl.ds(r0, BQ), :] = lax.dot(
                q_scr[pl.ds(r0, BQ), :], wo_ref[:, :],
                preferred_element_type=jnp.float32)
            return 0

        lax.fori_loop(0, NBLK, out_step, 0)

    out = pl.pallas_call(
        body,
        out_shape=jax.ShapeDtypeStruct((S, DM), jnp.float32),
        in_specs=[pl.BlockSpec(memory_space=pltpu.VMEM)] * 5,
        out_specs=pl.BlockSpec(memory_space=pltpu.VMEM),
        scratch_shapes=[
            pltpu.VMEM((2, BK, DM), jnp.bfloat16),
            pltpu.VMEM((2, BK, DM), jnp.bfloat16),
            pltpu.VMEM((S, DM), jnp.bfloat16),
            pltpu.VMEM((3, BQ, BK), jnp.float32),
            pltpu.SemaphoreType.DMA((4,)),
            pltpu.SemaphoreType.DMA((4,)),
            pltpu.SemaphoreType.DMA((4,)),
        ],
        compiler_params=pltpu.CompilerParams(
            collective_id=0,
            vmem_limit_bytes=60 * 1024 * 1024,
        ),
    )(x2, wq, k2, v2, wo)

    return out.reshape(1, S, DM)


# device time: 71892 ns/iter; 1.1959x vs baseline; 1.0320x over previous
import jax
import jax.numpy as jnp
from jax import lax
from jax.experimental import pallas as pl
from jax.experimental.pallas import tpu as pltpu

N_DEV = 4
S = 2048
H = 8
DH = 128
DM = H * DH
W = 128
BQ = 256
BK = BQ + 2 * W
NBLK = S // BQ
SCALE = 0.08838834764831843
NEG = -1e9


def kernel(x, Wq, K_ext, V_ext, Wo):
    x2 = x[0].astype(jnp.bfloat16)
    k2 = K_ext[0].reshape(S, DM).astype(jnp.bfloat16)
    v2 = V_ext[0].reshape(S, DM).astype(jnp.bfloat16)
    wq = (Wq * SCALE).astype(jnp.bfloat16)
    wo = Wo.astype(jnp.bfloat16)

    def body(x_ref, wq_ref, k_ref, v_ref, wo_ref, out_ref,
             edge_k, edge_v, q_scr, mask_scr,
             send_sems, recv_sems, copy_sems):
        p = lax.axis_index("i")
        left = lax.rem(p + N_DEV - 1, N_DEV)
        right = lax.rem(p + 1, N_DEV)

        barrier = pltpu.get_barrier_semaphore()
        for nbr in (left, right):
            pl.semaphore_signal(barrier, inc=1, device_id=(nbr,),
                                device_id_type=pl.DeviceIdType.MESH,)
        pl.semaphore_wait(barrier, 2)

        plan = [
            (k_ref.at[pl.ds(0, W), :], edge_k.at[1, pl.ds(BK - W, W), :], left),
            (k_ref.at[pl.ds(S - W, W), :], edge_k.at[0, pl.ds(0, W), :], right),
            (v_ref.at[pl.ds(0, W), :], edge_v.at[1, pl.ds(BK - W, W), :], left),
            (v_ref.at[pl.ds(S - W, W), :], edge_v.at[0, pl.ds(0, W), :], right),
        ]
        rdmas = []
        for i, (src, dst, tgt) in enumerate(plan):
            r = pltpu.make_async_remote_copy(
                src_ref=src, dst_ref=dst,
                send_sem=send_sems.at[i], recv_sem=recv_sems.at[i],
                device_id=(tgt,), device_id_type=pl.DeviceIdType.MESH,
            )
            r.start()
            rdmas.append(r)

        lcopies = []
        lplan = [
            (k_ref.at[pl.ds(0, BK - W), :], edge_k.at[0, pl.ds(W, BK - W), :]),
            (k_ref.at[pl.ds(S - BK + W, BK - W), :], edge_k.at[1, pl.ds(0, BK - W), :]),
            (v_ref.at[pl.ds(0, BK - W), :], edge_v.at[0, pl.ds(W, BK - W), :]),
            (v_ref.at[pl.ds(S - BK + W, BK - W), :], edge_v.at[1, pl.ds(0, BK - W), :]),
        ]
        for i, (src, dst) in enumerate(lplan):
            c = pltpu.make_async_copy(src, dst, copy_sems.at[i])
            c.start()
            lcopies.append(c)

        for rb in range(NBLK):
            r0 = rb * BQ
            qv = lax.dot(x_ref[pl.ds(r0, BQ), :], wq_ref[:, :],
                         preferred_element_type=jnp.float32)
            q_scr[pl.ds(r0, BQ), :] = qv.astype(jnp.bfloat16)

        ii = lax.broadcasted_iota(jnp.int32, (BQ, BK), 0)
        jj = lax.broadcasted_iota(jnp.int32, (BQ, BK), 1)
        band = jnp.where((jj >= ii) & (jj - ii <= 2 * W), 0.0, NEG)
        is_first = (p == 0)
        is_last = (p == N_DEV - 1)
        mask_scr[0, :, :] = band
        mask_scr[1, :, :] = band + jnp.where(is_first & (jj < W), NEG, 0.0)
        mask_scr[2, :, :] = band + jnp.where(is_last & (jj >= BK - W), NEG, 0.0)

        for c in lcopies:
            c.wait()
        for r in rdmas:
            r.wait()

        ones = jnp.ones((BK, DH), jnp.bfloat16)

        def attn_block(r0, kblk, vblk, mask, c0):
            qblk = q_scr[pl.ds(r0, BQ), c0:c0 + DH]
            s = lax.dot_general(
                qblk, kblk, (((1,), (1,)), ((), ())),
                preferred_element_type=jnp.float32)
            e = jnp.exp(s + mask).astype(jnp.bfloat16)
            denom = lax.dot_general(
                e, ones, (((1,), (0,)), ((), ())),
                preferred_element_type=jnp.float32)
            ctx = lax.dot_general(
                e, vblk, (((1,), (0,)), ((), ())),
                preferred_element_type=jnp.float32)
            q_scr[pl.ds(r0, BQ), c0:c0 + DH] = (ctx / denom).astype(jnp.bfloat16)

        for h in range(H):
            c0 = h * DH

            attn_block(0, edge_k[0, :, c0:c0 + DH], edge_v[0, :, c0:c0 + DH],
                       mask_scr[1, :, :], c0)

            for qb in range(1, NBLK - 1):
                r0 = qb * BQ
                attn_block(r0,
                           k_ref[pl.ds(r0 - W, BK), c0:c0 + DH],
                           v_ref[pl.ds(r0 - W, BK), c0:c0 + DH],
                           mask_scr[0, :, :], c0)

            attn_block((NBLK - 1) * BQ,
                       edge_k[1, :, c0:c0 + DH], edge_v[1, :, c0:c0 + DH],
                       mask_scr[2, :, :], c0)

        for rb in range(NBLK):
            r0 = rb * BQ
            out_ref[pl.ds(r0, BQ), :] = lax.dot(
                q_scr[pl.ds(r0, BQ), :], wo_ref[:, :],
                preferred_element_type=jnp.float32)

    out = pl.pallas_call(
        body,
        out_shape=jax.ShapeDtypeStruct((S, DM), jnp.float32),
        in_specs=[pl.BlockSpec(memory_space=pltpu.VMEM)] * 5,
        out_specs=pl.BlockSpec(memory_space=pltpu.VMEM),
        scratch_shapes=[
            pltpu.VMEM((2, BK, DM), jnp.bfloat16),
            pltpu.VMEM((2, BK, DM), jnp.bfloat16),
            pltpu.VMEM((S, DM), jnp.bfloat16),
            pltpu.VMEM((3, BQ, BK), jnp.float32),
            pltpu.SemaphoreType.DMA((4,)),
            pltpu.SemaphoreType.DMA((4,)),
            pltpu.SemaphoreType.DMA((4,)),
        ],
        compiler_params=pltpu.CompilerParams(
            collective_id=0,
            vmem_limit_bytes=60 * 1024 * 1024,
        ),
    )(x2, wq, k2, v2, wo)

    return out.reshape(1, S, DM)


# device time: 63600 ns/iter; 1.3518x vs baseline; 1.1304x over previous
import jax
import jax.numpy as jnp
from jax import lax
from jax.experimental import pallas as pl
from jax.experimental.pallas import tpu as pltpu

N_DEV = 4
S = 2048
H = 8
DH = 128
DM = H * DH
W = 128
BQ = 256
BK = BQ + 2 * W
NBLK = S // BQ
SCALE = 0.08838834764831843
NEG = -1e9


def kernel(x, Wq, K_ext, V_ext, Wo):
    x2 = x[0].astype(jnp.bfloat16)
    k2 = K_ext[0].reshape(S, DM).astype(jnp.bfloat16)
    v2 = V_ext[0].reshape(S, DM).astype(jnp.bfloat16)
    wq = (Wq * SCALE).astype(jnp.bfloat16)
    wo = Wo.astype(jnp.bfloat16)

    def body(x_ref, wq_ref, k_ref, v_ref, wo_ref, out_ref,
             edge_k, edge_v, q_scr,
             send_sems, recv_sems, copy_sems):
        p = lax.axis_index("i")
        left = lax.rem(p + N_DEV - 1, N_DEV)
        right = lax.rem(p + 1, N_DEV)

        barrier = pltpu.get_barrier_semaphore()
        for nbr in (left, right):
            pl.semaphore_signal(barrier, inc=1, device_id=(nbr,),
                                device_id_type=pl.DeviceIdType.MESH,)
        pl.semaphore_wait(barrier, 2)

        plan = [
            (k_ref.at[pl.ds(0, W), :], edge_k.at[1, pl.ds(BK - W, W), :], left),
            (k_ref.at[pl.ds(S - W, W), :], edge_k.at[0, pl.ds(0, W), :], right),
            (v_ref.at[pl.ds(0, W), :], edge_v.at[1, pl.ds(BK - W, W), :], left),
            (v_ref.at[pl.ds(S - W, W), :], edge_v.at[0, pl.ds(0, W), :], right),
        ]
        rdmas = []
        for i, (src, dst, tgt) in enumerate(plan):
            r = pltpu.make_async_remote_copy(
                src_ref=src, dst_ref=dst,
                send_sem=send_sems.at[i], recv_sem=recv_sems.at[i],
                device_id=(tgt,), device_id_type=pl.DeviceIdType.MESH,
            )
            r.start()
            rdmas.append(r)

        lcopies = []
        lplan = [
            (k_ref.at[pl.ds(0, BK - W), :], edge_k.at[0, pl.ds(W, BK - W), :]),
            (k_ref.at[pl.ds(S - BK + W, BK - W), :], edge_k.at[1, pl.ds(0, BK - W), :]),
            (v_ref.at[pl.ds(0, BK - W), :], edge_v.at[0, pl.ds(W, BK - W), :]),
            (v_ref.at[pl.ds(S - BK + W, BK - W), :], edge_v.at[1, pl.ds(0, BK - W), :]),
        ]
        for i, (src, dst) in enumerate(lplan):
            c = pltpu.make_async_copy(src, dst, copy_sems.at[i])
            c.start()
            lcopies.append(c)

        for rb in range(NBLK):
            r0 = rb * BQ
            qv = lax.dot(x_ref[pl.ds(r0, BQ), :], wq_ref[:, :],
                         preferred_element_type=jnp.float32)
            q_scr[pl.ds(r0, BQ), :] = qv.astype(jnp.bfloat16)

        is_first = (p == 0)
        is_last = (p == N_DEV - 1)

        for c in lcopies:
            c.wait()
        for r in rdmas:
            r.wait()

        def attn_block(r0, kblk, vblk, c0, edge=None):
            qblk = q_scr[pl.ds(r0, BQ), c0:c0 + DH]
            s = lax.dot_general(
                qblk, kblk, (((1,), (1,)), ((), ())),
                preferred_element_type=jnp.float32)
            ii = lax.broadcasted_iota(jnp.int32, (BQ, BK), 0)
            jj = lax.broadcasted_iota(jnp.int32, (BQ, BK), 1)
            band = (jj >= ii) & (jj - ii <= 2 * W)
            if edge == "first":
                band = band & ~(is_first & (jj < W))
            elif edge == "last":
                band = band & ~(is_last & (jj >= BK - W))
            e = jnp.exp(jnp.where(band, s, NEG))
            denom = jnp.sum(e, axis=1, keepdims=True)
            ctx = lax.dot_general(
                e.astype(jnp.bfloat16), vblk, (((1,), (0,)), ((), ())),
                preferred_element_type=jnp.float32)
            ctx = ctx * (1.0 / denom)
            q_scr[pl.ds(r0, BQ), c0:c0 + DH] = ctx.astype(jnp.bfloat16)

        for h in range(H):
            c0 = h * DH

            attn_block(0, edge_k[0, :, c0:c0 + DH], edge_v[0, :, c0:c0 + DH],
                       c0, edge="first")

            for qb in range(1, NBLK - 1):
                r0 = qb * BQ
                attn_block(r0,
                           k_ref[pl.ds(r0 - W, BK), c0:c0 + DH],
                           v_ref[pl.ds(r0 - W, BK), c0:c0 + DH],
                           c0)

            attn_block((NBLK - 1) * BQ,
                       edge_k[1, :, c0:c0 + DH], edge_v[1, :, c0:c0 + DH],
                       c0, edge="last")

        for rb in range(NBLK):
            r0 = rb * BQ
            out_ref[pl.ds(r0, BQ), :] = lax.dot(
                q_scr[pl.ds(r0, BQ), :], wo_ref[:, :],
                preferred_element_type=jnp.float32).astype(jnp.bfloat16)

    out = pl.pallas_call(
        body,
        out_shape=jax.ShapeDtypeStruct((S, DM), jnp.bfloat16),
        in_specs=[pl.BlockSpec(memory_space=pltpu.VMEM)] * 5,
        out_specs=pl.BlockSpec(memory_space=pltpu.VMEM),
        scratch_shapes=[
            pltpu.VMEM((2, BK, DM), jnp.bfloat16),
            pltpu.VMEM((2, BK, DM), jnp.bfloat16),
            pltpu.VMEM((S, DM), jnp.bfloat16),
            pltpu.SemaphoreType.DMA((4,)),
            pltpu.SemaphoreType.DMA((4,)),
            pltpu.SemaphoreType.DMA((4,)),
        ],
        compiler_params=pltpu.CompilerParams(
            collective_id=0,
            vmem_limit_bytes=60 * 1024 * 1024,
        ),
    )(x2, wq, k2, v2, wo)

    return out.reshape(1, S, DM)


# device time: 57140 ns/iter; 1.5046x vs baseline; 1.1131x over previous
import jax
import jax.numpy as jnp
from jax import lax
from jax.experimental import pallas as pl
from jax.experimental.pallas import tpu as pltpu

N_DEV = 4
S = 2048
H = 8
DH = 128
DM = H * DH
W = 128
BQ = 256
BK = BQ + 2 * W
NBLK = S // BQ
SCALE = 0.08838834764831843
NEG = -1e9


def kernel(x, Wq, K_ext, V_ext, Wo):
    x2 = x[0]
    k2 = K_ext[0].reshape(S, DM).astype(jnp.bfloat16)
    v2 = V_ext[0].reshape(S, DM).astype(jnp.bfloat16)
    wq = Wq
    wo = Wo

    def body(x_ref, wq_ref, k_ref, v_ref, wo_ref, out_ref,
             edge_k, edge_v, q_scr,
             send_sems, recv_sems, copy_sems):
        p = lax.axis_index("i")
        left = lax.rem(p + N_DEV - 1, N_DEV)
        right = lax.rem(p + 1, N_DEV)

        barrier = pltpu.get_barrier_semaphore()
        for nbr in (left, right):
            pl.semaphore_signal(barrier, inc=1, device_id=(nbr,),
                                device_id_type=pl.DeviceIdType.MESH,)
        pl.semaphore_wait(barrier, 2)

        plan = [
            (k_ref.at[pl.ds(0, W), :], edge_k.at[1, pl.ds(BK - W, W), :], left),
            (k_ref.at[pl.ds(S - W, W), :], edge_k.at[0, pl.ds(0, W), :], right),
            (v_ref.at[pl.ds(0, W), :], edge_v.at[1, pl.ds(BK - W, W), :], left),
            (v_ref.at[pl.ds(S - W, W), :], edge_v.at[0, pl.ds(0, W), :], right),
        ]
        rdmas = []
        for i, (src, dst, tgt) in enumerate(plan):
            r = pltpu.make_async_remote_copy(
                src_ref=src, dst_ref=dst,
                send_sem=send_sems.at[i], recv_sem=recv_sems.at[i],
                device_id=(tgt,), device_id_type=pl.DeviceIdType.MESH,
            )
            r.start()
            rdmas.append(r)

        lcopies = []
        lplan = [
            (k_ref.at[pl.ds(0, BK - W), :], edge_k.at[0, pl.ds(W, BK - W), :]),
            (k_ref.at[pl.ds(S - BK + W, BK - W), :], edge_k.at[1, pl.ds(0, BK - W), :]),
            (v_ref.at[pl.ds(0, BK - W), :], edge_v.at[0, pl.ds(W, BK - W), :]),
            (v_ref.at[pl.ds(S - BK + W, BK - W), :], edge_v.at[1, pl.ds(0, BK - W), :]),
        ]
        for i, (src, dst) in enumerate(lplan):
            c = pltpu.make_async_copy(src, dst, copy_sems.at[i])
            c.start()
            lcopies.append(c)

        for rb in range(NBLK):
            r0 = rb * BQ
            qv = lax.dot(x_ref[pl.ds(r0, BQ), :], wq_ref[:, :],
                         preferred_element_type=jnp.float32)
            q_scr[pl.ds(r0, BQ), :] = (qv * SCALE).astype(jnp.bfloat16)

        is_first = (p == 0)
        is_last = (p == N_DEV - 1)

        for c in lcopies:
            c.wait()
        for r in rdmas:
            r.wait()

        def attn_block(r0, kblk, vblk, c0, edge=None):
            qblk = q_scr[pl.ds(r0, BQ), c0:c0 + DH]
            s = lax.dot_general(
                qblk, kblk, (((1,), (1,)), ((), ())),
                preferred_element_type=jnp.float32)
            ii = lax.broadcasted_iota(jnp.int32, (BQ, BK), 0)
            jj = lax.broadcasted_iota(jnp.int32, (BQ, BK), 1)
            band = (jj >= ii) & (jj - ii <= 2 * W)
            if edge == "first":
                band = band & ~(is_first & (jj < W))
            elif edge == "last":
                band = band & ~(is_last & (jj >= BK - W))
            e = jnp.exp(jnp.where(band, s, NEG))
            denom = jnp.sum(e, axis=1, keepdims=True)
            ctx = lax.dot_general(
                e.astype(jnp.bfloat16), vblk, (((1,), (0,)), ((), ())),
                preferred_element_type=jnp.float32)
            ctx = ctx * (1.0 / denom)
            q_scr[pl.ds(r0, BQ), c0:c0 + DH] = ctx.astype(jnp.bfloat16)

        for h in range(H):
            c0 = h * DH

            attn_block(0, edge_k[0, :, c0:c0 + DH], edge_v[0, :, c0:c0 + DH],
                       c0, edge="first")

            for qb in range(1, NBLK - 1):
                r0 = qb * BQ
                attn_block(r0,
                           k_ref[pl.ds(r0 - W, BK), c0:c0 + DH],
                           v_ref[pl.ds(r0 - W, BK), c0:c0 + DH],
                           c0)

            attn_block((NBLK - 1) * BQ,
                       edge_k[1, :, c0:c0 + DH], edge_v[1, :, c0:c0 + DH],
                       c0, edge="last")

        for rb in range(NBLK):
            r0 = rb * BQ
            out_ref[pl.ds(r0, BQ), :] = lax.dot(
                q_scr[pl.ds(r0, BQ), :].astype(jnp.float32), wo_ref[:, :],
                preferred_element_type=jnp.float32).astype(jnp.bfloat16)

    out = pl.pallas_call(
        body,
        out_shape=jax.ShapeDtypeStruct((S, DM), jnp.bfloat16),
        in_specs=[pl.BlockSpec(memory_space=pltpu.VMEM)] * 5,
        out_specs=pl.BlockSpec(memory_space=pltpu.VMEM),
        scratch_shapes=[
            pltpu.VMEM((2, BK, DM), jnp.bfloat16),
            pltpu.VMEM((2, BK, DM), jnp.bfloat16),
            pltpu.VMEM((S, DM), jnp.bfloat16),
            pltpu.SemaphoreType.DMA((4,)),
            pltpu.SemaphoreType.DMA((4,)),
            pltpu.SemaphoreType.DMA((4,)),
        ],
        compiler_params=pltpu.CompilerParams(
            collective_id=0,
            vmem_limit_bytes=60 * 1024 * 1024,
        ),
    )(x2, wq, k2, v2, wo)

    return out.reshape(1, S, DM)


# device time: 54734 ns/iter; 1.5707x vs baseline; 1.0440x over previous
import jax
import jax.numpy as jnp
from jax import lax
from jax.experimental import pallas as pl
from jax.experimental.pallas import tpu as pltpu

N_DEV = 4
S = 2048
H = 8
DH = 128
DM = H * DH
W = 128
BQ = 256
BK = BQ + 2 * W
NBLK = S // BQ
SCALE = 0.08838834764831843
NEG = -1e9


def kernel(x, Wq, K_ext, V_ext, Wo):
    x2 = x[0]
    k2 = K_ext[0].reshape(S, DM).astype(jnp.bfloat16)
    v2 = V_ext[0].reshape(S, DM).astype(jnp.bfloat16)
    wq = Wq
    wo = Wo

    def body(x_ref, wq_ref, k_ref, v_ref, wo_ref, out_ref,
             edge_k, edge_v, q_scr,
             send_sems, recv_sems, copy_sems):
        p = lax.axis_index("i")
        left = lax.rem(p + N_DEV - 1, N_DEV)
        right = lax.rem(p + 1, N_DEV)

        barrier = pltpu.get_barrier_semaphore()
        for nbr in (left, right):
            pl.semaphore_signal(barrier, inc=1, device_id=(nbr,),
                                device_id_type=pl.DeviceIdType.MESH,)
        pl.semaphore_wait(barrier, 2)

        plan = [
            (k_ref.at[pl.ds(0, W), :], edge_k.at[1, pl.ds(BK - W, W), :], left),
            (k_ref.at[pl.ds(S - W, W), :], edge_k.at[0, pl.ds(0, W), :], right),
            (v_ref.at[pl.ds(0, W), :], edge_v.at[1, pl.ds(BK - W, W), :], left),
            (v_ref.at[pl.ds(S - W, W), :], edge_v.at[0, pl.ds(0, W), :], right),
        ]
        rdmas = []
        for i, (src, dst, tgt) in enumerate(plan):
            r = pltpu.make_async_remote_copy(
                src_ref=src, dst_ref=dst,
                send_sem=send_sems.at[i], recv_sem=recv_sems.at[i],
                device_id=(tgt,), device_id_type=pl.DeviceIdType.MESH,
            )
            r.start()
            rdmas.append(r)

        lcopies = []
        lplan = [
            (k_ref.at[pl.ds(0, BK - W), :], edge_k.at[0, pl.ds(W, BK - W), :]),
            (k_ref.at[pl.ds(S - BK + W, BK - W), :], edge_k.at[1, pl.ds(0, BK - W), :]),
            (v_ref.at[pl.ds(0, BK - W), :], edge_v.at[0, pl.ds(W, BK - W), :]),
            (v_ref.at[pl.ds(S - BK + W, BK - W), :], edge_v.at[1, pl.ds(0, BK - W), :]),
        ]
        for i, (src, dst) in enumerate(lplan):
            c = pltpu.make_async_copy(src, dst, copy_sems.at[i])
            c.start()
            lcopies.append(c)

        for rb in range(NBLK):
            r0 = rb * BQ
            qv = lax.dot(x_ref[pl.ds(r0, BQ), :], wq_ref[:, :],
                         preferred_element_type=jnp.float32)
            q_scr[pl.ds(r0, BQ), :] = (qv * SCALE).astype(jnp.bfloat16)

        is_first = (p == 0)
        is_last = (p == N_DEV - 1)

        def attn_block(r0, kblk, vblk, c0, edge=None):
            qblk = q_scr[pl.ds(r0, BQ), c0:c0 + DH]
            s = lax.dot_general(
                qblk, kblk, (((1,), (1,)), ((), ())),
                preferred_element_type=jnp.float32)
            ii = lax.broadcasted_iota(jnp.int32, (BQ, BK), 0)
            jj = lax.broadcasted_iota(jnp.int32, (BQ, BK), 1)
            band = (jj >= ii) & (jj - ii <= 2 * W)
            if edge == "first":
                band = band & ~(is_first & (jj < W))
            elif edge == "last":
                band = band & ~(is_last & (jj >= BK - W))
            e = jnp.exp(jnp.where(band, s, NEG).astype(jnp.bfloat16))
            denom = jnp.sum(e, axis=1, keepdims=True,
                            dtype=jnp.float32)
            ctx = lax.dot_general(
                e, vblk, (((1,), (0,)), ((), ())),
                preferred_element_type=jnp.float32)
            ctx = ctx * (1.0 / denom)
            q_scr[pl.ds(r0, BQ), c0:c0 + DH] = ctx.astype(jnp.bfloat16)

        for h in range(H):
            c0 = h * DH
            for qb in range(1, NBLK - 1):
                r0 = qb * BQ
                attn_block(r0,
                           k_ref[pl.ds(r0 - W, BK), c0:c0 + DH],
                           v_ref[pl.ds(r0 - W, BK), c0:c0 + DH],
                           c0)

        for c in lcopies:
            c.wait()
        for r in rdmas:
            r.wait()

        for h in range(H):
            c0 = h * DH
            attn_block(0, edge_k[0, :, c0:c0 + DH], edge_v[0, :, c0:c0 + DH],
                       c0, edge="first")
            attn_block((NBLK - 1) * BQ,
                       edge_k[1, :, c0:c0 + DH], edge_v[1, :, c0:c0 + DH],
                       c0, edge="last")

        for rb in range(NBLK):
            r0 = rb * BQ
            out_ref[pl.ds(r0, BQ), :] = lax.dot(
                q_scr[pl.ds(r0, BQ), :].astype(jnp.float32), wo_ref[:, :],
                preferred_element_type=jnp.float32).astype(jnp.bfloat16)

    out = pl.pallas_call(
        body,
        out_shape=jax.ShapeDtypeStruct((S, DM), jnp.bfloat16),
        in_specs=[pl.BlockSpec(memory_space=pltpu.VMEM)] * 5,
        out_specs=pl.BlockSpec(memory_space=pltpu.VMEM),
        scratch_shapes=[
            pltpu.VMEM((2, BK, DM), jnp.bfloat16),
            pltpu.VMEM((2, BK, DM), jnp.bfloat16),
            pltpu.VMEM((S, DM), jnp.bfloat16),
            pltpu.SemaphoreType.DMA((4,)),
            pltpu.SemaphoreType.DMA((4,)),
            pltpu.SemaphoreType.DMA((4,)),
        ],
        compiler_params=pltpu.CompilerParams(
            collective_id=0,
            vmem_limit_bytes=60 * 1024 * 1024,
        ),
    )(x2, wq, k2, v2, wo)

    return out.reshape(1, S, DM)


# device time: 53433 ns/iter; 1.6090x vs baseline; 1.0243x over previous
import jax
import jax.numpy as jnp
from jax import lax
from jax.experimental import pallas as pl
from jax.experimental.pallas import tpu as pltpu

N_DEV = 4
S = 2048
H = 8
DH = 128
DM = H * DH
W = 128
BQ = 256
NBLK = S // BQ
AQ = 512
AK = AQ + 2 * W
ANB = S // AQ
SCALE = 0.08838834764831843
NEG = -1e9


def kernel(x, Wq, K_ext, V_ext, Wo):
    x2 = x[0]
    k2 = K_ext[0].reshape(S, DM).astype(jnp.bfloat16)
    v2 = V_ext[0].reshape(S, DM).astype(jnp.bfloat16)
    wq = Wq
    wo = Wo

    def body(x_ref, wq_ref, k_ref, v_ref, wo_ref, out_ref,
             edge_k, edge_v, q_scr,
             send_sems, recv_sems, copy_sems):
        p = lax.axis_index("i")
        left = lax.rem(p + N_DEV - 1, N_DEV)
        right = lax.rem(p + 1, N_DEV)

        barrier = pltpu.get_barrier_semaphore()
        for nbr in (left, right):
            pl.semaphore_signal(barrier, inc=1, device_id=(nbr,),
                                device_id_type=pl.DeviceIdType.MESH,)
        pl.semaphore_wait(barrier, 2)

        plan = [
            (k_ref.at[pl.ds(0, W), :], edge_k.at[1, pl.ds(AK - W, W), :], left),
            (k_ref.at[pl.ds(S - W, W), :], edge_k.at[0, pl.ds(0, W), :], right),
            (v_ref.at[pl.ds(0, W), :], edge_v.at[1, pl.ds(AK - W, W), :], left),
            (v_ref.at[pl.ds(S - W, W), :], edge_v.at[0, pl.ds(0, W), :], right),
        ]
        rdmas = []
        for i, (src, dst, tgt) in enumerate(plan):
            r = pltpu.make_async_remote_copy(
                src_ref=src, dst_ref=dst,
                send_sem=send_sems.at[i], recv_sem=recv_sems.at[i],
                device_id=(tgt,), device_id_type=pl.DeviceIdType.MESH,
            )
            r.start()
            rdmas.append(r)

        lcopies = []
        lplan = [
            (k_ref.at[pl.ds(0, AK - W), :], edge_k.at[0, pl.ds(W, AK - W), :]),
            (k_ref.at[pl.ds(S - AK + W, AK - W), :], edge_k.at[1, pl.ds(0, AK - W), :]),
            (v_ref.at[pl.ds(0, AK - W), :], edge_v.at[0, pl.ds(W, AK - W), :]),
            (v_ref.at[pl.ds(S - AK + W, AK - W), :], edge_v.at[1, pl.ds(0, AK - W), :]),
        ]
        for i, (src, dst) in enumerate(lplan):
            c = pltpu.make_async_copy(src, dst, copy_sems.at[i])
            c.start()
            lcopies.append(c)

        for rb in range(NBLK):
            r0 = rb * BQ
            qv = lax.dot(x_ref[pl.ds(r0, BQ), :], wq_ref[:, :],
                         preferred_element_type=jnp.float32)
            q_scr[pl.ds(r0, BQ), :] = (qv * SCALE).astype(jnp.bfloat16)

        is_first = (p == 0)
        is_last = (p == N_DEV - 1)

        def attn_block(r0, kblk, vblk, c0, edge=None):
            qblk = q_scr[pl.ds(r0, AQ), c0:c0 + DH]
            s = lax.dot_general(
                qblk, kblk, (((1,), (1,)), ((), ())),
                preferred_element_type=jnp.float32)
            ii = lax.broadcasted_iota(jnp.int32, (AQ, AK), 0)
            jj = lax.broadcasted_iota(jnp.int32, (AQ, AK), 1)
            band = (jj >= ii) & (jj - ii <= 2 * W)
            if edge == "first":
                band = band & ~(is_first & (jj < W))
            elif edge == "last":
                band = band & ~(is_last & (jj >= AK - W))
            e = jnp.exp(jnp.where(band, s, NEG).astype(jnp.bfloat16))
            denom = jnp.sum(e, axis=1, keepdims=True,
                            dtype=jnp.float32)
            ctx = lax.dot_general(
                e, vblk, (((1,), (0,)), ((), ())),
                preferred_element_type=jnp.float32)
            ctx = ctx * (1.0 / denom)
            q_scr[pl.ds(r0, AQ), c0:c0 + DH] = ctx.astype(jnp.bfloat16)

        for h in range(H):
            c0 = h * DH
            for qb in range(1, ANB - 1):
                r0 = qb * AQ
                attn_block(r0,
                           k_ref[pl.ds(r0 - W, AK), c0:c0 + DH],
                           v_ref[pl.ds(r0 - W, AK), c0:c0 + DH],
                           c0)

        for c in lcopies:
            c.wait()
        for r in rdmas:
            r.wait()

        for h in range(H):
            c0 = h * DH
            attn_block(0, edge_k[0, :, c0:c0 + DH], edge_v[0, :, c0:c0 + DH],
                       c0, edge="first")
            attn_block((ANB - 1) * AQ,
                       edge_k[1, :, c0:c0 + DH], edge_v[1, :, c0:c0 + DH],
                       c0, edge="last")

        for rb in range(NBLK):
            r0 = rb * BQ
            out_ref[pl.ds(r0, BQ), :] = lax.dot(
                q_scr[pl.ds(r0, BQ), :].astype(jnp.float32), wo_ref[:, :],
                preferred_element_type=jnp.float32).astype(jnp.bfloat16)

    out = pl.pallas_call(
        body,
        out_shape=jax.ShapeDtypeStruct((S, DM), jnp.bfloat16),
        in_specs=[pl.BlockSpec(memory_space=pltpu.VMEM)] * 5,
        out_specs=pl.BlockSpec(memory_space=pltpu.VMEM),
        scratch_shapes=[
            pltpu.VMEM((2, AK, DM), jnp.bfloat16),
            pltpu.VMEM((2, AK, DM), jnp.bfloat16),
            pltpu.VMEM((S, DM), jnp.bfloat16),
            pltpu.SemaphoreType.DMA((4,)),
            pltpu.SemaphoreType.DMA((4,)),
            pltpu.SemaphoreType.DMA((4,)),
        ],
        compiler_params=pltpu.CompilerParams(
            collective_id=0,
            vmem_limit_bytes=60 * 1024 * 1024,
        ),
    )(x2, wq, k2, v2, wo)

    return out.reshape(1, S, DM)
